# baseline (speedup 1.0000x reference)
import functools

import jax
import jax.numpy as jnp
from jax.experimental import pallas as pl
from jax.experimental.pallas import tpu as pltpu

RATIO = 0.1
INT_MIN = -2147483648
INT_MAX = 2147483647


def _sortable(xi):
    return xi ^ ((xi >> 31) & jnp.int32(0x7FFFFFFF))


def _kwta_block(in_ref, out_ref, scratch_ref, *, k):
    rows = in_ref.shape[0]
    x = in_ref[...]
    xi = pltpu.bitcast(x, jnp.int32)
    scratch_ref[...] = _sortable(xi)

    kk = jnp.int32(k)
    nfeat = jnp.int32(in_ref.shape[1])

    def probe(d_b, lo_b, clo, hi_b, chi):
        d_s = d_b ^ jnp.int32(INT_MIN)
        m = scratch_ref[...]
        ge = m >= d_s
        cnt = jnp.sum(ge.astype(jnp.int32), axis=1, keepdims=True)
        mmin = jnp.min(jnp.where(ge, m, jnp.int32(INT_MAX)), axis=1,
                       keepdims=True)
        mmax = jnp.max(jnp.where(ge, jnp.int32(INT_MIN), m), axis=1,
                       keepdims=True)
        upd = cnt >= kk
        lo_b = jnp.where(upd, mmin ^ jnp.int32(INT_MIN), lo_b)
        clo = jnp.where(upd, cnt, clo)
        hi_b = jnp.where(upd, hi_b, mmax ^ jnp.int32(INT_MIN))
        chi = jnp.where(upd, chi, cnt)
        return lo_b, clo, hi_b, chi

    def not_done(state):
        i, lo_b, clo, hi_b, chi = state
        done = (clo == kk) | (lo_b == hi_b)
        return jnp.logical_and(i < 64, jnp.logical_not(jnp.all(done)))

    def body(state):
        i, lo_b, clo, hi_b, chi = state
        w = hi_b - lo_b
        wf = w.astype(jnp.float32)
        frac = (clo - kk + 1).astype(jnp.float32) / (clo - chi + 1).astype(
            jnp.float32)
        step_i = jnp.clip((wf * frac).astype(jnp.int32), 1, w)
        step_b = jnp.maximum(w >> 1, 1)
        use_bisect = (i % 2) == 1
        d_b = lo_b + jnp.where(use_bisect, step_b, step_i)
        d_b = jnp.where((clo == kk) | (lo_b == hi_b), lo_b, d_b)
        lo_b, clo, hi_b, chi = probe(d_b, lo_b, clo, hi_b, chi)
        return i + 1, lo_b, clo, hi_b, chi

    zero = jnp.zeros((rows, 1), jnp.int32)
    lo_b0, clo0, hi_b0, chi0 = probe(
        jnp.full((rows, 1), INT_MIN, jnp.int32),
        zero, zero + nfeat, zero - 1, zero)
    state = (jnp.int32(1), lo_b0, clo0, hi_b0, chi0)
    _, lo_b, _, _, _ = jax.lax.while_loop(not_done, body, state)

    thr_bits = _sortable(lo_b ^ jnp.int32(INT_MIN))
    thr = pltpu.bitcast(thr_bits, jnp.float32)
    out_ref[...] = jnp.where(x >= thr, x, jnp.float32(0.0))


def kernel(inputs):
    rows, features = inputs.shape
    k = max(int(round(RATIO * features)), 1)
    block_rows = 16
    grid = rows // block_rows
    return pl.pallas_call(
        functools.partial(_kwta_block, k=k),
        grid=(grid,),
        in_specs=[pl.BlockSpec((block_rows, features), lambda i: (i, 0))],
        out_specs=pl.BlockSpec((block_rows, features), lambda i: (i, 0)),
        out_shape=jax.ShapeDtypeStruct(inputs.shape, inputs.dtype),
        scratch_shapes=[pltpu.VMEM((block_rows, features), jnp.int32)],
    )(inputs)


# speedup vs baseline: 2.0669x; 2.0669x over previous
import functools

import jax
import jax.numpy as jnp
import numpy as np
from jax.experimental import pallas as pl
from jax.experimental.pallas import tpu as pltpu

RATIO = 0.1
INT_MIN = -2147483648
MAX_PASSES = 100
WARM_BIASED = int(np.float32(1.281552).view(np.int32)) ^ INT_MIN


def _sortable(xi):
    return xi ^ ((xi >> 31) & jnp.int32(0x7FFFFFFF))


def _kwta_block(in_ref, out_ref, scratch_ref, *, k):
    rows = in_ref.shape[0]
    x = in_ref[...]
    xi = pltpu.bitcast(x, jnp.int32)
    xi = jnp.where(x == 0.0, jnp.int32(0), xi)
    scratch_ref[...] = _sortable(xi)

    kk = jnp.int32(k)
    nfeat = jnp.int32(in_ref.shape[1])

    def count_ge(d_b):
        d_s = d_b ^ jnp.int32(INT_MIN)
        ge = scratch_ref[...] >= d_s
        return jnp.sum(ge.astype(jnp.int32), axis=1, keepdims=True)

    def update(d_b, cnt, lo_b, clo, hi_b, chi):
        upd = cnt >= kk
        lo_b = jnp.where(upd, d_b, lo_b)
        clo = jnp.where(upd, cnt, clo)
        hi_b = jnp.where(upd, hi_b, d_b - 1)
        chi = jnp.where(upd, chi, cnt)
        return lo_b, clo, hi_b, chi

    def not_done(state):
        i, lo_b, clo, hi_b, chi = state
        done = (clo == kk) | (lo_b == hi_b)
        return jnp.logical_and(i < MAX_PASSES,
                               jnp.logical_not(jnp.all(done)))

    def body(state):
        i, lo_b, clo, hi_b, chi = state
        w = hi_b - lo_b
        wf = w.astype(jnp.float32)
        frac = (clo - kk + 1).astype(jnp.float32) / (clo - chi + 1).astype(
            jnp.float32)
        step_i = jnp.clip((wf * frac).astype(jnp.int32), 1, w)
        step_b = jnp.maximum(w >> 1, 1)
        use_bisect = (i % 3) == 2
        d_b = lo_b + jnp.where(use_bisect, step_b, step_i)
        d_b = jnp.where(w < 0, jnp.int32(INT_MIN), d_b)
        done = (clo == kk) | (lo_b == hi_b)
        d_b = jnp.where(done, lo_b, d_b)
        cnt = count_ge(d_b)
        lo_b, clo, hi_b, chi = update(d_b, cnt, lo_b, clo, hi_b, chi)
        return i + 1, lo_b, clo, hi_b, chi

    zero = jnp.zeros((rows, 1), jnp.int32)
    d0 = jnp.full((rows, 1), WARM_BIASED, jnp.int32)
    cnt0 = count_ge(d0)
    lo_b, clo, hi_b, chi = update(d0, cnt0, zero, zero + nfeat, zero - 1,
                                  zero)
    state = (jnp.int32(1), lo_b, clo, hi_b, chi)
    _, lo_b, _, _, _ = jax.lax.while_loop(not_done, body, state)

    thr_bits = _sortable(lo_b ^ jnp.int32(INT_MIN))
    thr = pltpu.bitcast(thr_bits, jnp.float32)
    out_ref[...] = jnp.where(x >= thr, x, jnp.float32(0.0))


def kernel(inputs):
    rows, features = inputs.shape
    k = max(int(round(RATIO * features)), 1)
    block_rows = 16
    grid = rows // block_rows
    return pl.pallas_call(
        functools.partial(_kwta_block, k=k),
        grid=(grid,),
        in_specs=[pl.BlockSpec((block_rows, features), lambda i: (i, 0))],
        out_specs=pl.BlockSpec((block_rows, features), lambda i: (i, 0)),
        out_shape=jax.ShapeDtypeStruct(inputs.shape, inputs.dtype),
        scratch_shapes=[pltpu.VMEM((block_rows, features), jnp.int32)],
    )(inputs)
